# baseline (speedup 1.0000x reference)
import jax
import jax.numpy as jnp
from jax.experimental import pallas as pl
from jax.experimental.pallas import tpu as pltpu

T_TILE = 512
NUM_HEADS = 16
D_HEAD = 128
D_MODEL = 2048
K_CHUNKS = 4
CHUNK = NUM_HEADS * D_HEAD // K_CHUNKS


def _body(emb_ref, idx_ref, p_ref, w_hbm, b_ref, out_ref,
          wbf_ref, wstage_ref, sem):
    pid = pl.program_id(0)
    emb = emb_ref[...]
    idx = idx_ref[...]
    p = p_ref[...]
    px0 = (p[:, 0:1] * emb[:, 0, :]).astype(jnp.bfloat16)
    px1 = (p[:, 1:2] * emb[:, 1, :]).astype(jnp.bfloat16)
    iota = jax.lax.broadcasted_iota(jnp.int32, (T_TILE, NUM_HEADS), 1)
    oh0 = (idx[:, 0:1] == iota)
    oh1 = (idx[:, 1:2] == iota)
    oh0b = oh0.astype(jnp.bfloat16)
    oh1b = oh1.astype(jnp.bfloat16)
    m = oh0.astype(jnp.float32) * p[:, 0:1] + oh1.astype(jnp.float32) * p[:, 1:2]
    acc = jnp.dot(m, b_ref[...], preferred_element_type=jnp.float32)

    @pl.when(pid == 0)
    def _start_w_copies():
        for k in range(K_CHUNKS):
            pltpu.make_async_copy(
                w_hbm.at[pl.ds(k * CHUNK, CHUNK), :],
                wstage_ref.at[k], sem.at[k]).start()

    for k in range(K_CHUNKS):
        @pl.when(pid == 0)
        def _land_chunk(k=k):
            pltpu.make_async_copy(
                w_hbm.at[pl.ds(k * CHUNK, CHUNK), :],
                wstage_ref.at[k], sem.at[k]).wait()
            wbf_ref[k * CHUNK:(k + 1) * CHUNK, :] = (
                wstage_ref[k].astype(jnp.bfloat16))

        for e in range(4 * k, 4 * k + 4, 2):
            x0 = oh0b[:, e:e + 1] * px0 + oh1b[:, e:e + 1] * px1
            x1 = oh0b[:, e + 1:e + 2] * px0 + oh1b[:, e + 1:e + 2] * px1
            xe = jnp.concatenate([x0, x1], axis=1)
            acc = acc + jnp.dot(xe, wbf_ref[e * D_HEAD:(e + 2) * D_HEAD, :],
                                preferred_element_type=jnp.float32)
    out_ref[...] = acc


def kernel(embedding, sel_idx, sel_probs, W, b):
    T = embedding.shape[0]
    wflat = W.reshape(NUM_HEADS * D_HEAD, D_MODEL)
    grid = (T // T_TILE,)
    return pl.pallas_call(
        _body,
        grid=grid,
        in_specs=[
            pl.BlockSpec((T_TILE, 2, D_HEAD), lambda t: (t, 0, 0)),
            pl.BlockSpec((T_TILE, 2), lambda t: (t, 0)),
            pl.BlockSpec((T_TILE, 2), lambda t: (t, 0)),
            pl.BlockSpec(memory_space=pl.ANY),
            pl.BlockSpec((NUM_HEADS, D_MODEL), lambda t: (0, 0)),
        ],
        out_specs=pl.BlockSpec((T_TILE, D_MODEL), lambda t: (t, 0)),
        out_shape=jax.ShapeDtypeStruct((T, D_MODEL), jnp.float32),
        scratch_shapes=[
            pltpu.VMEM((NUM_HEADS * D_HEAD, D_MODEL), jnp.bfloat16),
            pltpu.VMEM((K_CHUNKS, CHUNK, D_MODEL), jnp.float32),
            pltpu.SemaphoreType.DMA((K_CHUNKS,)),
        ],
    )(embedding, sel_idx.astype(jnp.int32), sel_probs, wflat, b)


# speedup vs baseline: 1.1167x; 1.1167x over previous
import jax
import jax.numpy as jnp
from jax.experimental import pallas as pl
from jax.experimental.pallas import tpu as pltpu

T_TILE = 1024
NUM_HEADS = 16
D_HEAD = 128
D_MODEL = 2048
GROUP = 8


def _body(emb_ref, idx_ref, p_ref, w_ref, b_ref, out_ref):
    emb = emb_ref[...]
    idx = idx_ref[...]
    p = p_ref[...]
    px0 = p[:, 0:1] * emb[:, 0, :]
    px1 = p[:, 1:2] * emb[:, 1, :]
    iota = jax.lax.broadcasted_iota(jnp.int32, (T_TILE, NUM_HEADS), 1)
    oh0 = (idx[:, 0:1] == iota)
    oh1 = (idx[:, 1:2] == iota)
    oh0b = oh0.astype(jnp.float32)
    oh1b = oh1.astype(jnp.float32)
    m = oh0.astype(jnp.float32) * p[:, 0:1] + oh1.astype(jnp.float32) * p[:, 1:2]
    out_ref[...] = jnp.dot(m, b_ref[...], preferred_element_type=jnp.float32)
    for e in range(0, NUM_HEADS, GROUP):
        xs = [oh0b[:, g:g + 1] * px0 + oh1b[:, g:g + 1] * px1
              for g in range(e, e + GROUP)]
        xe = xs[0] if GROUP == 1 else jnp.concatenate(xs, axis=1)
        out_ref[...] += jnp.dot(xe, w_ref[e * D_HEAD:(e + GROUP) * D_HEAD, :],
                                preferred_element_type=jnp.float32)


def kernel(embedding, sel_idx, sel_probs, W, b):
    T = embedding.shape[0]
    wflat = W.reshape(NUM_HEADS * D_HEAD, D_MODEL)
    grid = (T // T_TILE,)
    return pl.pallas_call(
        _body,
        grid=grid,
        in_specs=[
            pl.BlockSpec((T_TILE, 2, D_HEAD), lambda t: (t, 0, 0)),
            pl.BlockSpec((T_TILE, 2), lambda t: (t, 0)),
            pl.BlockSpec((T_TILE, 2), lambda t: (t, 0)),
            pl.BlockSpec((NUM_HEADS * D_HEAD, D_MODEL), lambda t: (0, 0)),
            pl.BlockSpec((NUM_HEADS, D_MODEL), lambda t: (0, 0)),
        ],
        out_specs=pl.BlockSpec((T_TILE, D_MODEL), lambda t: (t, 0)),
        out_shape=jax.ShapeDtypeStruct((T, D_MODEL), jnp.float32),
    )(embedding, sel_idx.astype(jnp.int32), sel_probs, wflat, b)
